# baseline (speedup 1.0000x reference)
import functools

import jax
import jax.numpy as jnp
from jax.experimental import pallas as pl
from jax.experimental.pallas import tpu as pltpu

_STACK = 4
_ROW_BLOCK = 256


def _vq_body(x_ref, proj_hbm, cb_hbm, out_ref,
             proj_ref, cbn_ref, sem):
    i = pl.program_id(0)

    @pl.when(i == 0)
    def _():
        cp = pltpu.make_async_copy(proj_hbm, proj_ref, sem)
        cp.start()
        cp.wait()
        cc = pltpu.make_async_copy(cb_hbm, cbn_ref, sem)
        cc.start()
        cc.wait()
        cb = cbn_ref[...]
        norm = jnp.sqrt(jnp.sum(cb * cb, axis=1, keepdims=True))
        cbn_ref[...] = cb / jnp.maximum(norm, 1e-12)

    xb = x_ref[0]
    xs = xb.reshape(_ROW_BLOCK, _STACK * xb.shape[1])
    p = jnp.dot(xs, proj_ref[...],
                preferred_element_type=jnp.float32)
    scores = jnp.dot(p, cbn_ref[...].T,
                     preferred_element_type=jnp.float32,
                     precision=jax.lax.Precision.HIGHEST)
    idx = jnp.argmax(scores, axis=1).astype(jnp.int32)
    per_b = out_ref.shape[1] // _ROW_BLOCK
    out_ref[i // per_b, pl.ds((i % per_b) * _ROW_BLOCK, _ROW_BLOCK)] = idx


@functools.partial(jax.jit, static_argnames=())
def kernel(x, proj, codebook):
    b, t, c = x.shape
    t_out = t // _STACK
    t_blk = _ROW_BLOCK * _STACK
    per_b = t // t_blk
    grid = b * per_b
    return pl.pallas_call(
        _vq_body,
        grid=(grid,),
        in_specs=[
            pl.BlockSpec((1, t_blk, c),
                         lambda i: (i // per_b, i % per_b, 0)),
            pl.BlockSpec(memory_space=pltpu.MemorySpace.HBM),
            pl.BlockSpec(memory_space=pltpu.MemorySpace.HBM),
        ],
        out_specs=pl.BlockSpec((b, t_out), lambda i: (0, 0)),
        out_shape=jax.ShapeDtypeStruct((b, t_out), jnp.int32),
        scratch_shapes=[
            pltpu.VMEM(proj.shape, jnp.float32),
            pltpu.VMEM(codebook.shape, jnp.float32),
            pltpu.SemaphoreType.DMA,
        ],
    )(x, proj, codebook)


# speedup vs baseline: 1.1057x; 1.1057x over previous
import functools

import jax
import jax.numpy as jnp
from jax.experimental import pallas as pl
from jax.experimental.pallas import tpu as pltpu

_STACK = 4
_ROW_BLOCK = 512


def _vq_body(x_ref, proj_ref, cb_ref, out_ref, cbn_ref):
    i = pl.program_id(0)

    @pl.when(i == 0)
    def _():
        cb = cb_ref[...]
        norm = jnp.sqrt(jnp.sum(cb * cb, axis=1, keepdims=True))
        cbn_ref[...] = cb / jnp.maximum(norm, 1e-12)

    xb = x_ref[0]
    xs = xb.reshape(_ROW_BLOCK, _STACK * xb.shape[1])
    p = jnp.dot(xs, proj_ref[...],
                preferred_element_type=jnp.float32)
    scores = jnp.dot(p, cbn_ref[...].T,
                     preferred_element_type=jnp.float32,
                     precision=jax.lax.Precision.HIGHEST)
    idx = jnp.argmax(scores, axis=1).astype(jnp.int32)
    per_b = out_ref.shape[1] // _ROW_BLOCK
    out_ref[i // per_b, pl.ds((i % per_b) * _ROW_BLOCK, _ROW_BLOCK)] = idx


@functools.partial(jax.jit, static_argnames=())
def kernel(x, proj, codebook):
    b, t, c = x.shape
    t_out = t // _STACK
    t_blk = _ROW_BLOCK * _STACK
    per_b = t // t_blk
    grid = b * per_b
    return pl.pallas_call(
        _vq_body,
        grid=(grid,),
        in_specs=[
            pl.BlockSpec((1, t_blk, c),
                         lambda i: (i // per_b, i % per_b, 0)),
            pl.BlockSpec(proj.shape, lambda i: (0, 0)),
            pl.BlockSpec(codebook.shape, lambda i: (0, 0)),
        ],
        out_specs=pl.BlockSpec((b, t_out), lambda i: (0, 0)),
        out_shape=jax.ShapeDtypeStruct((b, t_out), jnp.int32),
        scratch_shapes=[pltpu.VMEM(codebook.shape, jnp.float32)],
    )(x, proj, codebook)


# speedup vs baseline: 1.1754x; 1.0630x over previous
import functools

import jax
import jax.numpy as jnp
from jax.experimental import pallas as pl
from jax.experimental.pallas import tpu as pltpu

_STACK = 4
_ROW_BLOCK = 512


def _vq_body(x_ref, proj_ref, cb_ref, out_ref, cbn_ref):
    i = pl.program_id(0)

    @pl.when(i == 0)
    def _():
        cb = cb_ref[...]
        norm = jnp.sqrt(jnp.sum(cb * cb, axis=1, keepdims=True))
        cbn_ref[...] = cb / jnp.maximum(norm, 1e-12)

    xb = x_ref[0]
    xs = xb.reshape(_ROW_BLOCK, _STACK * xb.shape[1])
    p = jnp.dot(xs, proj_ref[...],
                preferred_element_type=jnp.float32)
    per_b = out_ref.shape[1] // _ROW_BLOCK
    half = _ROW_BLOCK // 2
    for h in range(2):
        scores = jnp.dot(p[h * half:(h + 1) * half], cbn_ref[...].T,
                         preferred_element_type=jnp.float32,
                         precision=jax.lax.Precision.HIGHEST)
        idx = jnp.argmax(scores, axis=1).astype(jnp.int32)
        out_ref[i // per_b,
                pl.ds((i % per_b) * _ROW_BLOCK + h * half, half)] = idx


@functools.partial(jax.jit, static_argnames=())
def kernel(x, proj, codebook):
    b, t, c = x.shape
    t_out = t // _STACK
    t_blk = _ROW_BLOCK * _STACK
    per_b = t // t_blk
    grid = b * per_b
    return pl.pallas_call(
        _vq_body,
        grid=(grid,),
        in_specs=[
            pl.BlockSpec((1, t_blk, c),
                         lambda i: (i // per_b, i % per_b, 0)),
            pl.BlockSpec(proj.shape, lambda i: (0, 0)),
            pl.BlockSpec(codebook.shape, lambda i: (0, 0)),
        ],
        out_specs=pl.BlockSpec((b, t_out), lambda i: (0, 0)),
        out_shape=jax.ShapeDtypeStruct((b, t_out), jnp.int32),
        scratch_shapes=[pltpu.VMEM(codebook.shape, jnp.float32)],
    )(x, proj, codebook)


# speedup vs baseline: 1.2584x; 1.0707x over previous
import functools

import jax
import jax.numpy as jnp
from jax.experimental import pallas as pl
from jax.experimental.pallas import tpu as pltpu

_STACK = 4
_ROW_BLOCK = 512


def _vq_body(x_ref, w_ref, out_ref, cbn_ref):
    i = pl.program_id(0)
    k = w_ref.shape[0] - cbn_ref.shape[0]

    @pl.when(i == 0)
    def _():
        cb = w_ref[k:, :]
        norm = jnp.sqrt(jnp.sum(cb * cb, axis=1, keepdims=True))
        cbn_ref[...] = cb / jnp.maximum(norm, 1e-12)

    xb = x_ref[0]
    xs = xb.reshape(_ROW_BLOCK, _STACK * xb.shape[1])
    p = jnp.dot(xs, w_ref[:k, :],
                preferred_element_type=jnp.float32)
    per_b = out_ref.shape[1] // _ROW_BLOCK
    half = _ROW_BLOCK // 2
    for h in range(2):
        scores = jnp.dot(p[h * half:(h + 1) * half], cbn_ref[...].T,
                         preferred_element_type=jnp.float32,
                         precision=jax.lax.Precision.HIGHEST)
        idx = jnp.argmax(scores, axis=1).astype(jnp.int32)
        out_ref[i // per_b,
                pl.ds((i % per_b) * _ROW_BLOCK + h * half, half)] = idx


@functools.partial(jax.jit, static_argnames=())
def kernel(x, proj, codebook):
    b, t, c = x.shape
    t_out = t // _STACK
    t_blk = _ROW_BLOCK * _STACK
    per_b = t // t_blk
    grid = b * per_b
    w = jnp.concatenate([proj, codebook], axis=0)
    return pl.pallas_call(
        _vq_body,
        grid=(grid,),
        in_specs=[
            pl.BlockSpec((1, t_blk, c),
                         lambda i: (i // per_b, i % per_b, 0)),
            pl.BlockSpec(w.shape, lambda i: (0, 0)),
        ],
        out_specs=pl.BlockSpec((b, t_out), lambda i: (0, 0)),
        out_shape=jax.ShapeDtypeStruct((b, t_out), jnp.int32),
        scratch_shapes=[pltpu.VMEM(codebook.shape, jnp.float32)],
    )(x, w)


# speedup vs baseline: 1.3626x; 1.0828x over previous
import functools

import jax
import jax.numpy as jnp
from jax.experimental import pallas as pl
from jax.experimental.pallas import tpu as pltpu

_STACK = 4
_ROW_BLOCK = 512


def _vq_body(x_ref, w_ref, out_ref, cbn_ref):
    i = pl.program_id(0)
    k = w_ref.shape[0] - cbn_ref.shape[1]

    @pl.when(i == 0)
    def _():
        cb = w_ref[k:, :]
        norm = jnp.sqrt(jnp.sum(cb * cb, axis=1, keepdims=True))
        cbn_ref[...] = (cb / jnp.maximum(norm, 1e-12)).T

    xb = x_ref[0]
    per_b = out_ref.shape[1] // _ROW_BLOCK
    half = _ROW_BLOCK // 2
    for h in range(2):
        xs = xb[h * half * _STACK:(h + 1) * half * _STACK]
        xs = xs.reshape(half, _STACK * xb.shape[1])
        p = jnp.dot(xs, w_ref[:k, :],
                    preferred_element_type=jnp.float32)
        scores = jnp.dot(p, cbn_ref[...],
                         preferred_element_type=jnp.float32,
                         precision=jax.lax.Precision.HIGHEST)
        idx = jnp.argmax(scores, axis=1).astype(jnp.int32)
        out_ref[i // per_b,
                pl.ds((i % per_b) * _ROW_BLOCK + h * half, half)] = idx


@functools.partial(jax.jit, static_argnames=())
def kernel(x, proj, codebook):
    b, t, c = x.shape
    t_out = t // _STACK
    t_blk = _ROW_BLOCK * _STACK
    per_b = t // t_blk
    grid = b * per_b
    w = jnp.concatenate([proj, codebook], axis=0)
    return pl.pallas_call(
        _vq_body,
        grid=(grid,),
        in_specs=[
            pl.BlockSpec((1, t_blk, c),
                         lambda i: (i // per_b, i % per_b, 0)),
            pl.BlockSpec(w.shape, lambda i: (0, 0)),
        ],
        out_specs=pl.BlockSpec((b, t_out), lambda i: (0, 0)),
        out_shape=jax.ShapeDtypeStruct((b, t_out), jnp.int32),
        scratch_shapes=[pltpu.VMEM(codebook.shape[::-1], jnp.float32)],
    )(x, w)
